# Initial kernel scaffold; baseline (speedup 1.0000x reference)
import functools

import jax
import jax.numpy as jnp
from jax import lax
from jax.experimental import pallas as pl
from jax.experimental.pallas import tpu as pltpu
from jax.experimental.pallas import tpu_sc as plsc

B = 100000
D = 128
C = 128
NW = 32
NCHUNKS = -(-B // C)
LAST_BASE = B - C
CPW = -(-NCHUNKS // NW)


def _sc_gather(idx, table):
    mesh = plsc.VectorSubcoreMesh(core_axis_name="c", subcore_axis_name="s")

    @functools.partial(
        pl.kernel,
        mesh=mesh,
        out_type=jax.ShapeDtypeStruct((B, D), jnp.float32),
        scratch_types=[
            pltpu.VMEM((C,), jnp.int32),
            pltpu.VMEM((C, D), jnp.float32),
            pltpu.SemaphoreType.DMA,
        ],
    )
    def k(idx_hbm, table_hbm, out_hbm, idx_v, rows_v, sem):
        wid = lax.axis_index("s") * 2 + lax.axis_index("c")

        def body(j, carry):
            chunk = wid + j * NW

            @pl.when(chunk < NCHUNKS)
            def _():
                base = jnp.minimum(chunk * C, LAST_BASE)
                pltpu.sync_copy(idx_hbm.at[pl.ds(base, C)], idx_v)
                pltpu.async_copy(table_hbm.at[idx_v], rows_v, sem).wait()
                pltpu.sync_copy(rows_v, out_hbm.at[pl.ds(base, C)])

            return carry

        lax.fori_loop(0, CPW, body, 0)

    return k(idx, table)


def kernel(atomic_numbers, table):
    idx = atomic_numbers.astype(jnp.int32)
    return _sc_gather(idx, table)


# speedup vs baseline: 1.6534x; 1.6534x over previous
import functools

import jax
import jax.numpy as jnp
from jax import lax
from jax.experimental import pallas as pl
from jax.experimental.pallas import tpu as pltpu
from jax.experimental.pallas import tpu_sc as plsc

B = 100000
D = 128
C = 128
NW = 32
NBUF = 4
NCHUNKS = -(-B // C)
LAST_BASE = B - C
CPW = -(-NCHUNKS // NW)
FULL_W = NCHUNKS - (CPW - 1) * NW


def _sc_gather(idx, table):
    mesh = plsc.VectorSubcoreMesh(core_axis_name="c", subcore_axis_name="s")

    @functools.partial(
        pl.kernel,
        mesh=mesh,
        out_type=jax.ShapeDtypeStruct((B, D), jnp.float32),
        scratch_types=[
            pltpu.VMEM((CPW, C), jnp.int32),
            pltpu.VMEM((NBUF, C, D), jnp.float32),
            pltpu.SemaphoreType.DMA,
            *[pltpu.SemaphoreType.DMA] * NBUF,
            *[pltpu.SemaphoreType.DMA] * NBUF,
        ],
    )
    def k(idx_hbm, table_hbm, out_hbm, idx_all, rows, sem_i,
          g0, g1, g2, g3, s0, s1, s2, s3):
        gsem = [g0, g1, g2, g3]
        ssem = [s0, s1, s2, s3]
        wid = lax.axis_index("s") * 2 + lax.axis_index("c")

        def base_of(j):
            return jnp.minimum((wid + j * NW) * C, LAST_BASE)

        def guarded(j, fn):
            def run():
                fn()

            if j < CPW - 1:
                run()
            else:
                pl.when(wid < FULL_W)(run)

        for j in range(CPW):
            guarded(j, lambda j=j: pltpu.async_copy(
                idx_hbm.at[pl.ds(base_of(j), C)], idx_all.at[j], sem_i))
        for j in range(CPW):
            guarded(j, lambda j=j: pltpu.make_async_copy(
                idx_hbm.at[pl.ds(0, C)], idx_all.at[j], sem_i).wait())

        for b in range(NBUF):
            pltpu.async_copy(table_hbm.at[idx_all.at[b]], rows.at[b],
                             gsem[b])

        for j in range(CPW):
            b = j % NBUF

            def step(j=j, b=b):
                pltpu.make_async_copy(table_hbm.at[idx_all.at[j]],
                                      rows.at[b], gsem[b]).wait()
                st = pltpu.async_copy(rows.at[b],
                                      out_hbm.at[pl.ds(base_of(j), C)],
                                      ssem[b])
                jn = j + NBUF
                if jn < CPW:
                    st.wait()
                    guarded(jn, lambda: pltpu.async_copy(
                        table_hbm.at[idx_all.at[jn]], rows.at[b],
                        gsem[b]))

            guarded(j, step)

        for j in range(max(0, CPW - NBUF), CPW):
            guarded(j, lambda j=j, b=j % NBUF: pltpu.make_async_copy(
                rows.at[b], out_hbm.at[pl.ds(base_of(j), C)],
                ssem[b]).wait())

    return k(idx, table)


def kernel(atomic_numbers, table):
    idx = atomic_numbers.astype(jnp.int32)
    return _sc_gather(idx, table)


# speedup vs baseline: 5.4479x; 3.2949x over previous
import functools

import jax
import jax.numpy as jnp
from jax import lax
from jax.experimental import pallas as pl
from jax.experimental.pallas import tpu as pltpu
from jax.experimental.pallas import tpu_sc as plsc

B = 100000
D = 128
C = 128
NW = 32
NBUF = 4
NCHUNKS = -(-B // C)
LAST_BASE = B - C
CPW = -(-NCHUNKS // NW)
FULL_W = NCHUNKS - (CPW - 1) * NW


def _sc_gather(idx, table):
    mesh = plsc.VectorSubcoreMesh(core_axis_name="c", subcore_axis_name="s")

    @functools.partial(
        pl.kernel,
        mesh=mesh,
        out_type=jax.ShapeDtypeStruct((B, D), jnp.float32),
        scratch_types=[
            pltpu.VMEM((CPW, C), jnp.int32),
            pltpu.VMEM((NBUF, C, D), jnp.float32),
            pltpu.VMEM_SHARED((119, D), jnp.float32),
            pltpu.SemaphoreType.DMA,
            *[pltpu.SemaphoreType.DMA] * NBUF,
            *[pltpu.SemaphoreType.DMA] * NBUF,
        ],
    )
    def k(idx_hbm, table_hbm, out_hbm, idx_all, rows, table_v, sem_i,
          g0, g1, g2, g3, s0, s1, s2, s3):
        gsem = [g0, g1, g2, g3]
        ssem = [s0, s1, s2, s3]
        wid = lax.axis_index("s") * 2 + lax.axis_index("c")

        def base_of(j):
            return jnp.minimum((wid + j * NW) * C, LAST_BASE)

        def guarded(j, fn):
            def run():
                fn()

            if j < CPW - 1:
                run()
            else:
                pl.when(wid < FULL_W)(run)

        pl.when(lax.axis_index("s") == 0)(
            lambda: pltpu.sync_copy(table_hbm, table_v))
        plsc.subcore_barrier()

        for j in range(CPW):
            guarded(j, lambda j=j: pltpu.async_copy(
                idx_hbm.at[pl.ds(base_of(j), C)], idx_all.at[j], sem_i))
        for j in range(CPW):
            guarded(j, lambda j=j: pltpu.make_async_copy(
                idx_hbm.at[pl.ds(0, C)], idx_all.at[j], sem_i).wait())

        for b in range(NBUF):
            pltpu.async_copy(table_v.at[idx_all.at[b]], rows.at[b],
                             gsem[b])

        for j in range(CPW):
            b = j % NBUF

            def step(j=j, b=b):
                pltpu.make_async_copy(table_v.at[idx_all.at[j]],
                                      rows.at[b], gsem[b]).wait()
                st = pltpu.async_copy(rows.at[b],
                                      out_hbm.at[pl.ds(base_of(j), C)],
                                      ssem[b])
                jn = j + NBUF
                if jn < CPW:
                    st.wait()
                    guarded(jn, lambda: pltpu.async_copy(
                        table_v.at[idx_all.at[jn]], rows.at[b],
                        gsem[b]))

            guarded(j, step)

        for j in range(max(0, CPW - NBUF), CPW):
            guarded(j, lambda j=j, b=j % NBUF: pltpu.make_async_copy(
                rows.at[b], out_hbm.at[pl.ds(base_of(j), C)],
                ssem[b]).wait())

    return k(idx, table)


def kernel(atomic_numbers, table):
    idx = atomic_numbers.astype(jnp.int32)
    return _sc_gather(idx, table)


# speedup vs baseline: 5.6614x; 1.0392x over previous
import functools

import jax
import jax.numpy as jnp
from jax import lax
from jax.experimental import pallas as pl
from jax.experimental.pallas import tpu as pltpu
from jax.experimental.pallas import tpu_sc as plsc

B = 100000
D = 128
C = 128
NW = 32
NBUF = 4
AHEAD = 2
NCHUNKS = -(-B // C)
LAST_BASE = B - C
CPW = -(-NCHUNKS // NW)
FULL_W = NCHUNKS - (CPW - 1) * NW
ROUNDS = -(-(CPW + AHEAD) // NBUF)


def _sc_gather(idx, table):
    mesh = plsc.VectorSubcoreMesh(core_axis_name="c", subcore_axis_name="s")

    @functools.partial(
        pl.kernel,
        mesh=mesh,
        out_type=jax.ShapeDtypeStruct((B, D), jnp.float32),
        scratch_types=[
            pltpu.VMEM((CPW, C), jnp.int32),
            pltpu.VMEM((NBUF, C, D), jnp.float32),
            pltpu.VMEM_SHARED((119, D), jnp.float32),
            pltpu.SemaphoreType.DMA,
            *[pltpu.SemaphoreType.DMA] * NBUF,
            *[pltpu.SemaphoreType.DMA] * NBUF,
        ],
    )
    def k(idx_hbm, table_hbm, out_hbm, idx_all, rows, table_v, sem_i,
          g0, g1, g2, g3, s0, s1, s2, s3):
        gsem = [g0, g1, g2, g3]
        ssem = [s0, s1, s2, s3]
        wid = lax.axis_index("s") * 2 + lax.axis_index("c")
        cnt = (CPW - 1) + (wid < FULL_W).astype(jnp.int32)

        def base_of(j):
            return jnp.minimum((wid + j * NW) * C, LAST_BASE)

        pl.when(lax.axis_index("s") == 0)(
            lambda: pltpu.sync_copy(table_hbm, table_v))
        plsc.subcore_barrier()

        def fire_idx(j, carry):
            pltpu.async_copy(idx_hbm.at[pl.ds(base_of(j), C)],
                             idx_all.at[j], sem_i)
            return carry

        def drain_idx(j, carry):
            pltpu.make_async_copy(idx_hbm.at[pl.ds(0, C)],
                                  idx_all.at[j], sem_i).wait()
            return carry

        lax.fori_loop(0, cnt, fire_idx, 0)
        lax.fori_loop(0, cnt, drain_idx, 0)

        def gather(j, b):
            return pltpu.make_async_copy(table_v.at[idx_all.at[j]],
                                         rows.at[b], gsem[b])

        def store(j, b):
            return pltpu.make_async_copy(rows.at[b],
                                         out_hbm.at[pl.ds(base_of(j), C)],
                                         ssem[b])

        for b in range(AHEAD):
            gather(b, b).start()

        def round_body(r, carry):
            for b in range(NBUF):
                j = r * NBUF + b
                bn = (b + AHEAD) % NBUF

                def part_a(j=j, b=b):
                    gather(j, b).wait()
                    store(j, b).start()

                def part_b(j=j, bn=bn):
                    store(j - AHEAD, bn).wait()

                def part_c(j=j, bn=bn):
                    gather(j + AHEAD, bn).start()

                pl.when(j < cnt)(part_a)
                pl.when((j >= AHEAD) & (j - AHEAD < cnt))(part_b)
                pl.when(j + AHEAD < cnt)(part_c)
            return carry

        lax.fori_loop(0, ROUNDS, round_body, 0)

    return k(idx, table)


def kernel(atomic_numbers, table):
    idx = atomic_numbers.astype(jnp.int32)
    return _sc_gather(idx, table)


# speedup vs baseline: 5.7584x; 1.0171x over previous
import functools

import jax
import jax.numpy as jnp
from jax import lax
from jax.experimental import pallas as pl
from jax.experimental.pallas import tpu as pltpu
from jax.experimental.pallas import tpu_sc as plsc

B = 100000
D = 128
C = 128
NW = 32
NBUF = 6
AHEAD = 3
NCHUNKS = -(-B // C)
LAST_BASE = B - C
CPW = -(-NCHUNKS // NW)
FULL_W = NCHUNKS - (CPW - 1) * NW
ROUNDS = -(-(CPW + AHEAD) // NBUF)


def _sc_gather(idx, table):
    mesh = plsc.VectorSubcoreMesh(core_axis_name="c", subcore_axis_name="s")

    @functools.partial(
        pl.kernel,
        mesh=mesh,
        out_type=jax.ShapeDtypeStruct((B, D), jnp.float32),
        scratch_types=[
            pltpu.VMEM((CPW, C), jnp.int32),
            pltpu.VMEM((NBUF, C, D), jnp.float32),
            pltpu.VMEM_SHARED((119, D), jnp.float32),
            pltpu.SemaphoreType.DMA,
            *[pltpu.SemaphoreType.DMA] * NBUF,
            *[pltpu.SemaphoreType.DMA] * NBUF,
        ],
    )
    def k(idx_hbm, table_hbm, out_hbm, idx_all, rows, table_v, sem_i,
          g0, g1, g2, g3, g4, g5, s0, s1, s2, s3, s4, s5):
        gsem = [g0, g1, g2, g3, g4, g5]
        ssem = [s0, s1, s2, s3, s4, s5]
        wid = lax.axis_index("s") * 2 + lax.axis_index("c")
        cnt = (CPW - 1) + (wid < FULL_W).astype(jnp.int32)

        def base_of(j):
            return jnp.minimum((wid + j * NW) * C, LAST_BASE)

        pl.when(lax.axis_index("s") == 0)(
            lambda: pltpu.sync_copy(table_hbm, table_v))
        plsc.subcore_barrier()

        def fire_idx(j, carry):
            pltpu.async_copy(idx_hbm.at[pl.ds(base_of(j), C)],
                             idx_all.at[j], sem_i)
            return carry

        def drain_idx(j, carry):
            pltpu.make_async_copy(idx_hbm.at[pl.ds(0, C)],
                                  idx_all.at[j], sem_i).wait()
            return carry

        lax.fori_loop(0, cnt, fire_idx, 0)
        lax.fori_loop(0, cnt, drain_idx, 0)

        def gather(j, b):
            return pltpu.make_async_copy(table_v.at[idx_all.at[j]],
                                         rows.at[b], gsem[b])

        def store(j, b):
            return pltpu.make_async_copy(rows.at[b],
                                         out_hbm.at[pl.ds(base_of(j), C)],
                                         ssem[b])

        for b in range(AHEAD):
            gather(b, b).start()

        def round_body(r, carry):
            for b in range(NBUF):
                j = r * NBUF + b
                bn = (b + AHEAD) % NBUF

                def part_a(j=j, b=b):
                    gather(j, b).wait()
                    store(j, b).start()

                def part_b(j=j, bn=bn):
                    store(j - AHEAD, bn).wait()

                def part_c(j=j, bn=bn):
                    gather(j + AHEAD, bn).start()

                pl.when(j < cnt)(part_a)
                pl.when((j >= AHEAD) & (j - AHEAD < cnt))(part_b)
                pl.when(j + AHEAD < cnt)(part_c)
            return carry

        lax.fori_loop(0, ROUNDS, round_body, 0)

    return k(idx, table)


def kernel(atomic_numbers, table):
    idx = atomic_numbers.astype(jnp.int32)
    return _sc_gather(idx, table)
